# baseline (speedup 1.0000x reference)
import functools

import jax
import jax.numpy as jnp
from jax import lax
from jax.experimental import pallas as pl
from jax.experimental.pallas import tpu as pltpu

_ALPHA = 0.4
_TB = 128
_TE = 128
_TW2 = 256
_TW = 512


def _deg_kernel(rows_ref, dr_ref, dc_ref, *, n_tiles):
    i = pl.program_id(0)
    sub = lax.broadcasted_iota(jnp.int32, (_TB, _TW), 0)

    def body(t, acc):
        rl = rows_ref[:, pl.ds(t * _TW, _TW)] - i * _TB
        return acc + jnp.sum((sub == rl).astype(jnp.float32),
                             axis=1, keepdims=True)

    deg = lax.fori_loop(0, n_tiles, body, jnp.zeros((_TB, 1), jnp.float32))
    dinv = jnp.where(deg > 0.0, lax.rsqrt(deg), 0.0)
    dr_ref[...] = dinv
    dc_ref[...] = jnp.transpose(dinv)


def _densify_kernel(starts_ref, skey_ref, attr_ref, dr_ref, dc_ref, a_ref, *,
                    nb):
    i = pl.program_id(0)
    sub = lax.broadcasted_iota(jnp.int32, (_TB, _TW2), 0)
    dr = dr_ref[...]

    def window(p, off):
        sk = skey_ref[:, pl.ds(off, _TW2)]
        aw = attr_ref[:, pl.ds(off, _TW2)]
        ok = (sk >> 14) == p
        rl = (sk >> 7) & (_TB - 1)
        cl = sk & (_TB - 1)
        lhs = jnp.where((sub == rl) & ok, aw, 0.0)
        rhs_t = (sub == cl).astype(jnp.float32)
        return lax.dot_general(
            lhs, rhs_t,
            dimension_numbers=(((1,), (1,)), ((), ())),
            preferred_element_type=jnp.float32)

    overflow = jnp.int32(0)
    for j in range(nb):
        p = i * nb + j
        s0 = starts_ref[p]
        s1 = starts_ref[p + 1]
        w0 = (s0 // _TB) * _TB
        overflow = overflow | (s1 - w0 > _TW2).astype(jnp.int32)
        acc = window(p, w0)
        out = acc * dr * dc_ref[:, j * _TB:(j + 1) * _TB]
        a_ref[:, j * _TB:(j + 1) * _TB] = out.astype(a_ref.dtype)

    @pl.when(overflow > 0)
    def _():
        for j in range(nb):
            p = i * nb + j
            s0 = starts_ref[p]
            s1 = starts_ref[p + 1]
            w0 = (s0 // _TB) * _TB
            nw = jnp.where(s1 > s0, (s1 - w0 + _TW2 - 1) // _TW2, 0)
            extra = lax.fori_loop(
                1, nw, lambda k, a: a + window(p, w0 + k * _TW2),
                jnp.zeros((_TB, _TB), jnp.float32))
            sl = slice(j * _TB, (j + 1) * _TB)
            out = extra * dr * dc_ref[:, sl]
            a_ref[:, sl] = (a_ref[:, sl].astype(jnp.float32) +
                            out).astype(a_ref.dtype)


def _horner_kernel(a_row_ref, a_full_ref, b_ref, *, theta):
    i = pl.program_id(0)
    bm, bn = b_ref.shape
    a_row = a_row_ref[...]
    acc = jnp.dot(a_row, a_full_ref[...],
                  preferred_element_type=jnp.float32)
    acc = acc + a_row.astype(jnp.float32)
    eye = ((lax.broadcasted_iota(jnp.int32, (bm, bn), 0) + i * bm) ==
           lax.broadcasted_iota(jnp.int32, (bm, bn), 1))
    b_ref[...] = (theta * acc +
                  jnp.where(eye, theta, 0.0)).astype(b_ref.dtype)


def _indices_kernel(o_ref):
    p = pl.program_id(0)
    i = pl.program_id(1)
    _, bm, bn = o_ref.shape
    ri = lax.broadcasted_iota(jnp.int32, (1, bm, bn), 1) + i * bm
    ci = lax.broadcasted_iota(jnp.int32, (1, bm, bn), 2)
    o_ref[...] = jnp.where(p == 0, ri, ci)


def _final_kernel(a_row_ref, b_full_ref, s_ref, *, alpha):
    i = pl.program_id(0)
    bm, bn = s_ref.shape
    acc = jnp.dot(a_row_ref[...], b_full_ref[...],
                  preferred_element_type=jnp.float32)
    eye = ((lax.broadcasted_iota(jnp.int32, (bm, bn), 0) + i * bm) ==
           lax.broadcasted_iota(jnp.int32, (bm, bn), 1))
    s_ref[...] = acc + jnp.where(eye, alpha, 0.0)


def kernel(x, edge_index, edge_attr):
    n = x.shape[0]
    e = edge_attr.shape[0]
    nb = n // _TB
    npairs = nb * nb
    theta = _ALPHA * (1.0 - _ALPHA)
    ep = -(-e // _TW) * _TW + _TW

    rows = edge_index[0].astype(jnp.int32)
    cols = edge_index[1].astype(jnp.int32)

    pair = (rows >> 7) * nb + (cols >> 7)
    skey = (pair << 14) | ((rows & (_TB - 1)) << 7) | (cols & (_TB - 1))
    skey_s, attr_s = lax.sort((skey, edge_attr.astype(jnp.float32)),
                              num_keys=1)

    pad_key = jnp.full((ep - e,), jnp.int32(1) << 30, jnp.int32)
    skey_pad = jnp.concatenate([skey_s, pad_key]).reshape(1, ep)
    attr_pad = jnp.concatenate(
        [attr_s, jnp.zeros((ep - e,), jnp.float32)]).reshape(1, ep)
    rows_pad = jnp.concatenate(
        [rows, jnp.full((ep - e,), n, jnp.int32)]).reshape(1, ep)

    bounds = (jnp.arange(npairs + 1, dtype=jnp.int32) << 14)
    starts = jnp.sum(skey_s[None, :] < bounds[:, None],
                     axis=1).astype(jnp.int32)

    dinv_r, dinv_c = pl.pallas_call(
        functools.partial(_deg_kernel, n_tiles=ep // _TW),
        out_shape=(jax.ShapeDtypeStruct((n, 1), jnp.float32),
                   jax.ShapeDtypeStruct((1, n), jnp.float32)),
        grid=(nb,),
        in_specs=[pl.BlockSpec((1, ep), lambda i: (0, 0))],
        out_specs=(pl.BlockSpec((_TB, 1), lambda i: (i, 0)),
                   pl.BlockSpec((1, _TB), lambda i: (0, i))),
        compiler_params=pltpu.CompilerParams(
            dimension_semantics=("parallel",)),
    )(rows_pad)

    a_bf = pl.pallas_call(
        functools.partial(_densify_kernel, nb=nb),
        out_shape=jax.ShapeDtypeStruct((n, n), jnp.bfloat16),
        grid_spec=pltpu.PrefetchScalarGridSpec(
            num_scalar_prefetch=1,
            grid=(nb,),
            in_specs=[
                pl.BlockSpec((1, ep), lambda i, s: (0, 0)),
                pl.BlockSpec((1, ep), lambda i, s: (0, 0)),
                pl.BlockSpec((_TB, 1), lambda i, s: (i, 0)),
                pl.BlockSpec((1, n), lambda i, s: (0, 0)),
            ],
            out_specs=pl.BlockSpec((_TB, n), lambda i, s: (i, 0))),
        compiler_params=pltpu.CompilerParams(
            dimension_semantics=("parallel",)),
    )(starts, skey_pad, attr_pad, dinv_r, dinv_c)

    bm = max(n // 2, _TB)
    gm = n // bm
    mm_params = pltpu.CompilerParams(
        dimension_semantics=("parallel",))

    b_bf = pl.pallas_call(
        functools.partial(_horner_kernel, theta=theta),
        out_shape=jax.ShapeDtypeStruct((n, n), jnp.bfloat16),
        grid=(gm,),
        in_specs=[pl.BlockSpec((bm, n), lambda i: (i, 0)),
                  pl.BlockSpec((n, n), lambda i: (0, 0))],
        out_specs=pl.BlockSpec((bm, n), lambda i: (i, 0)),
        compiler_params=mm_params,
    )(a_bf, a_bf)

    s_mat = pl.pallas_call(
        functools.partial(_final_kernel, alpha=_ALPHA),
        out_shape=jax.ShapeDtypeStruct((n, n), jnp.float32),
        grid=(gm,),
        in_specs=[pl.BlockSpec((bm, n), lambda i: (i, 0)),
                  pl.BlockSpec((n, n), lambda i: (0, 0))],
        out_specs=pl.BlockSpec((bm, n), lambda i: (i, 0)),
        compiler_params=mm_params,
    )(a_bf, b_bf)

    idx = pl.pallas_call(
        _indices_kernel,
        out_shape=jax.ShapeDtypeStruct((2, n, n), jnp.int32),
        grid=(2, gm),
        out_specs=pl.BlockSpec((1, bm, n), lambda p, i: (p, i, 0)),
        compiler_params=pltpu.CompilerParams(
            dimension_semantics=("parallel", "parallel")),
    )()
    return idx.reshape(2, n * n), s_mat.reshape(-1)


# speedup vs baseline: 1.0207x; 1.0207x over previous
import functools

import jax
import jax.numpy as jnp
from jax import lax
from jax.experimental import pallas as pl
from jax.experimental.pallas import tpu as pltpu

_ALPHA = 0.4
_TB = 128
_TE = 128
_TW2 = 256
_TW = 512


def _deg_kernel(rows_ref, dr_ref, dc_ref, idx_ref, *, n_tiles):
    i = pl.program_id(0)
    sub = lax.broadcasted_iota(jnp.int32, (_TB, _TW), 0)

    def body(t, acc):
        rl = rows_ref[:, pl.ds(t * _TW, _TW)] - i * _TB
        return acc + jnp.sum((sub == rl).astype(jnp.float32),
                             axis=1, keepdims=True)

    deg = lax.fori_loop(0, n_tiles, body, jnp.zeros((_TB, 1), jnp.float32))
    dinv = jnp.where(deg > 0.0, lax.rsqrt(deg), 0.0)
    dr_ref[...] = dinv
    dc_ref[...] = jnp.transpose(dinv)
    _, bm, bn = idx_ref.shape
    idx_ref[0] = lax.broadcasted_iota(jnp.int32, (bm, bn), 0) + i * bm
    idx_ref[1] = lax.broadcasted_iota(jnp.int32, (bm, bn), 1)


def _densify_kernel(starts_ref, skey_ref, attr_ref, dr_ref, dc_ref, a_ref, *,
                    nb):
    i = pl.program_id(0)
    sub = lax.broadcasted_iota(jnp.int32, (_TB, _TW2), 0)
    dr = dr_ref[...]

    def window(p, off):
        sk = skey_ref[:, pl.ds(off, _TW2)]
        aw = attr_ref[:, pl.ds(off, _TW2)]
        ok = (sk >> 14) == p
        rl = (sk >> 7) & (_TB - 1)
        cl = sk & (_TB - 1)
        lhs = jnp.where((sub == rl) & ok, aw, 0.0)
        rhs_t = (sub == cl).astype(jnp.float32)
        return lax.dot_general(
            lhs, rhs_t,
            dimension_numbers=(((1,), (1,)), ((), ())),
            preferred_element_type=jnp.float32)

    overflow = jnp.int32(0)
    for j in range(nb):
        p = i * nb + j
        s0 = starts_ref[p]
        s1 = starts_ref[p + 1]
        w0 = (s0 // _TB) * _TB
        overflow = overflow | (s1 - w0 > _TW2).astype(jnp.int32)
        acc = window(p, w0)
        out = acc * dr * dc_ref[:, j * _TB:(j + 1) * _TB]
        a_ref[:, j * _TB:(j + 1) * _TB] = out.astype(a_ref.dtype)

    @pl.when(overflow > 0)
    def _():
        for j in range(nb):
            p = i * nb + j
            s0 = starts_ref[p]
            s1 = starts_ref[p + 1]
            w0 = (s0 // _TB) * _TB
            nw = jnp.where(s1 > s0, (s1 - w0 + _TW2 - 1) // _TW2, 0)
            extra = lax.fori_loop(
                1, nw, lambda k, a: a + window(p, w0 + k * _TW2),
                jnp.zeros((_TB, _TB), jnp.float32))
            sl = slice(j * _TB, (j + 1) * _TB)
            out = extra * dr * dc_ref[:, sl]
            a_ref[:, sl] = (a_ref[:, sl].astype(jnp.float32) +
                            out).astype(a_ref.dtype)


def _horner_kernel(a_row_ref, a_full_ref, b_ref, *, theta):
    i = pl.program_id(0)
    bm, bn = b_ref.shape
    a_row = a_row_ref[...]
    acc = jnp.dot(a_row, a_full_ref[...],
                  preferred_element_type=jnp.float32)
    acc = acc + a_row.astype(jnp.float32)
    eye = ((lax.broadcasted_iota(jnp.int32, (bm, bn), 0) + i * bm) ==
           lax.broadcasted_iota(jnp.int32, (bm, bn), 1))
    b_ref[...] = (theta * acc +
                  jnp.where(eye, theta, 0.0)).astype(b_ref.dtype)


def _final_kernel(a_row_ref, b_full_ref, s_ref, *, alpha):
    i = pl.program_id(0)
    bm, bn = s_ref.shape
    acc = jnp.dot(a_row_ref[...], b_full_ref[...],
                  preferred_element_type=jnp.float32)
    eye = ((lax.broadcasted_iota(jnp.int32, (bm, bn), 0) + i * bm) ==
           lax.broadcasted_iota(jnp.int32, (bm, bn), 1))
    s_ref[...] = acc + jnp.where(eye, alpha, 0.0)


def kernel(x, edge_index, edge_attr):
    n = x.shape[0]
    e = edge_attr.shape[0]
    nb = n // _TB
    npairs = nb * nb
    theta = _ALPHA * (1.0 - _ALPHA)
    ep = -(-e // _TW) * _TW + _TW

    rows = edge_index[0].astype(jnp.int32)
    cols = edge_index[1].astype(jnp.int32)

    pair = (rows >> 7) * nb + (cols >> 7)
    skey = (pair << 14) | ((rows & (_TB - 1)) << 7) | (cols & (_TB - 1))
    skey_s, attr_s = lax.sort((skey, edge_attr.astype(jnp.float32)),
                              num_keys=1)

    pad_key = jnp.full((ep - e,), jnp.int32(1) << 30, jnp.int32)
    skey_pad = jnp.concatenate([skey_s, pad_key]).reshape(1, ep)
    attr_pad = jnp.concatenate(
        [attr_s, jnp.zeros((ep - e,), jnp.float32)]).reshape(1, ep)
    rows_pad = jnp.concatenate(
        [rows, jnp.full((ep - e,), n, jnp.int32)]).reshape(1, ep)

    bounds = (jnp.arange(npairs + 1, dtype=jnp.int32) << 14)
    starts = jnp.sum(skey_s[None, :] < bounds[:, None],
                     axis=1).astype(jnp.int32)

    dinv_r, dinv_c, idx = pl.pallas_call(
        functools.partial(_deg_kernel, n_tiles=ep // _TW),
        out_shape=(jax.ShapeDtypeStruct((n, 1), jnp.float32),
                   jax.ShapeDtypeStruct((1, n), jnp.float32),
                   jax.ShapeDtypeStruct((2, n, n), jnp.int32)),
        grid=(nb,),
        in_specs=[pl.BlockSpec((1, ep), lambda i: (0, 0))],
        out_specs=(pl.BlockSpec((_TB, 1), lambda i: (i, 0)),
                   pl.BlockSpec((1, _TB), lambda i: (0, i)),
                   pl.BlockSpec((2, _TB, n), lambda i: (0, i, 0))),
        compiler_params=pltpu.CompilerParams(
            dimension_semantics=("parallel",)),
    )(rows_pad)

    a_bf = pl.pallas_call(
        functools.partial(_densify_kernel, nb=nb),
        out_shape=jax.ShapeDtypeStruct((n, n), jnp.bfloat16),
        grid_spec=pltpu.PrefetchScalarGridSpec(
            num_scalar_prefetch=1,
            grid=(nb,),
            in_specs=[
                pl.BlockSpec((1, ep), lambda i, s: (0, 0)),
                pl.BlockSpec((1, ep), lambda i, s: (0, 0)),
                pl.BlockSpec((_TB, 1), lambda i, s: (i, 0)),
                pl.BlockSpec((1, n), lambda i, s: (0, 0)),
            ],
            out_specs=pl.BlockSpec((_TB, n), lambda i, s: (i, 0))),
        compiler_params=pltpu.CompilerParams(
            dimension_semantics=("parallel",)),
    )(starts, skey_pad, attr_pad, dinv_r, dinv_c)

    bm = max(n // 2, _TB)
    gm = n // bm
    mm_params = pltpu.CompilerParams(
        dimension_semantics=("parallel",))

    b_bf = pl.pallas_call(
        functools.partial(_horner_kernel, theta=theta),
        out_shape=jax.ShapeDtypeStruct((n, n), jnp.bfloat16),
        grid=(gm,),
        in_specs=[pl.BlockSpec((bm, n), lambda i: (i, 0)),
                  pl.BlockSpec((n, n), lambda i: (0, 0))],
        out_specs=pl.BlockSpec((bm, n), lambda i: (i, 0)),
        compiler_params=mm_params,
    )(a_bf, a_bf)

    s_mat = pl.pallas_call(
        functools.partial(_final_kernel, alpha=_ALPHA),
        out_shape=jax.ShapeDtypeStruct((n, n), jnp.float32),
        grid=(gm,),
        in_specs=[pl.BlockSpec((bm, n), lambda i: (i, 0)),
                  pl.BlockSpec((n, n), lambda i: (0, 0))],
        out_specs=pl.BlockSpec((bm, n), lambda i: (i, 0)),
        compiler_params=mm_params,
    )(a_bf, b_bf)

    return idx.reshape(2, n * n), s_mat.reshape(-1)
